# baseline (speedup 1.0000x reference)
import math

import jax
import jax.numpy as jnp
import numpy as np
from jax import lax
from jax.experimental import pallas as pl
from jax.experimental.shard_map import shard_map
from jax.sharding import Mesh, PartitionSpec as P

_N_STREAMS = 2


def _feat_mm_body(x_ref, w_ref, y_ref):
    y_ref[...] = lax.dot_general(
        x_ref[...], w_ref[...], (((1,), (1,)), ((), ())),
        preferred_element_type=jnp.float32)


def _agg_body(a0, a1, y_ref, b_ref, al_ref, o_ref):
    alpha = al_ref[0, 0]
    for q, a_ref in enumerate((a0, a1)):
        acc = lax.dot_general(
            a_ref[0], y_ref[...], (((1,), (0,)), ((), ())),
            preferred_element_type=jnp.float32)
        h = acc + b_ref[...]
        o_ref[q] = jnp.where(h >= 0.0, h, alpha * h)


def _gcn_layer(x, adj_rows, w, b, alpha, bm_feat=2000, bm_agg=200):
    n, d_in = x.shape
    rows = adj_rows.shape[0]
    d_out = w.shape[0]
    y = pl.pallas_call(
        _feat_mm_body,
        grid=(n // bm_feat,),
        in_specs=[
            pl.BlockSpec((bm_feat, d_in), lambda i: (i, 0)),
            pl.BlockSpec((d_out, d_in), lambda i: (0, 0)),
        ],
        out_specs=pl.BlockSpec((bm_feat, d_out), lambda i: (i, 0)),
        out_shape=jax.ShapeDtypeStruct((n, d_out), jnp.float32),
    )(x, w)
    ns = _N_STREAMS
    rows_per_stream = rows // ns
    adj3 = adj_rows.reshape(ns, rows_per_stream, n)
    adj_specs = [
        pl.BlockSpec((1, bm_agg, n), lambda i, q=q: (q, i, 0))
        for q in range(ns)
    ]
    h = pl.pallas_call(
        _agg_body,
        grid=(math.ceil(rows_per_stream / bm_agg),),
        in_specs=adj_specs + [
            pl.BlockSpec((n, d_out), lambda i: (0, 0)),
            pl.BlockSpec((1, d_out), lambda i: (0, 0)),
            pl.BlockSpec((1, 1), lambda i: (0, 0)),
        ],
        out_specs=pl.BlockSpec((ns, bm_agg, d_out), lambda i: (0, i, 0)),
        out_shape=jax.ShapeDtypeStruct((ns, rows_per_stream, d_out), jnp.float32),
    )(*([adj3] * ns), y, b.reshape(1, -1), alpha.reshape(1, 1))
    return h.reshape(rows, d_out)


def kernel(features, seq1, adj, b1, W1, a1, b2, W2, a2, sparse):
    del seq1, sparse
    x = features[0]
    adj2d = adj[0]
    tpus = [d for d in jax.devices() if d.platform == "tpu"]
    if len(tpus) >= 2:
        mesh = Mesh(np.array(tpus[:2]), ("x",))

        def _sharded(xs, adj_loc, w1, bb1, aa1, w2, bb2, aa2):
            h1_loc = _gcn_layer(xs, adj_loc, w1, bb1, aa1)
            h1 = lax.all_gather(h1_loc, "x", axis=0, tiled=True)
            h2_loc = _gcn_layer(h1, adj_loc, w2, bb2, aa2)
            return h2_loc

        h2 = shard_map(
            _sharded, mesh=mesh,
            in_specs=(P(), P("x", None), P(), P(), P(), P(), P(), P()),
            out_specs=P("x", None),
            check_rep=False,
        )(x, adj2d, W1, b1, a1, W2, b2, a2)
    else:
        h1 = _gcn_layer(x, adj2d, W1, b1, a1)
        h2 = _gcn_layer(h1, adj2d, W2, b2, a2)
    return h2[None]


# speedup vs baseline: 3.5531x; 3.5531x over previous
import math

import jax
import jax.numpy as jnp
from jax import lax
from jax.experimental import pallas as pl
from jax.experimental.pallas import tpu as pltpu

_N_STREAMS = 2
_BM_FEAT = 1000
_BM_AGG = 200


def _layer_body(x_ref, w_ref, b_ref, al_ref, a0, a1, o_ref, y_scr):
    i = pl.program_id(0)
    nf = y_scr.shape[0] // x_ref.shape[0]

    @pl.when(i < nf)
    def _feat():
        row = pl.multiple_of(i * x_ref.shape[0], x_ref.shape[0])
        y_scr[pl.ds(row, x_ref.shape[0]), :] = lax.dot_general(
            x_ref[...], w_ref[...], (((1,), (1,)), ((), ())),
            preferred_element_type=jnp.float32)

    @pl.when(i >= nf)
    def _agg():
        alpha = al_ref[0, 0]
        for q, a_ref in enumerate((a0, a1)):
            acc = lax.dot_general(
                a_ref[0], y_scr[...], (((1,), (0,)), ((), ())),
                preferred_element_type=jnp.float32)
            h = acc + b_ref[...]
            o_ref[q] = jnp.where(h >= 0.0, h, alpha * h)


def _gcn_layer(x, adj3, w, b, alpha):
    n, d_in = x.shape
    ns, rows_per_stream, _ = adj3.shape
    d_out = w.shape[0]
    nf = n // _BM_FEAT
    na = math.ceil(rows_per_stream / _BM_AGG)
    grid = (nf + na,)

    def _x_map(i):
        return (jnp.minimum(i, nf - 1), 0)

    def _adj_map_for(q):
        def _m(i):
            return (q, jnp.maximum(i - nf, 0), 0)
        return _m

    def _out_map(i):
        return (0, jnp.maximum(i - nf, 0), 0)

    h = pl.pallas_call(
        _layer_body,
        grid=grid,
        in_specs=[
            pl.BlockSpec((_BM_FEAT, d_in), _x_map),
            pl.BlockSpec((d_out, d_in), lambda i: (0, 0)),
            pl.BlockSpec((1, d_out), lambda i: (0, 0)),
            pl.BlockSpec((1, 1), lambda i: (0, 0)),
        ] + [
            pl.BlockSpec((1, _BM_AGG, n), _adj_map_for(q))
            for q in range(ns)
        ],
        out_specs=pl.BlockSpec((ns, _BM_AGG, d_out), _out_map),
        out_shape=jax.ShapeDtypeStruct((ns, rows_per_stream, d_out), jnp.float32),
        scratch_shapes=[pltpu.VMEM((n, d_out), jnp.float32)],
    )(x, w, b.reshape(1, -1), alpha.reshape(1, 1), *([adj3] * ns))
    return h.reshape(ns * rows_per_stream, d_out)


def kernel(features, seq1, adj, b1, W1, a1, b2, W2, a2, sparse):
    del seq1, sparse
    x = features[0]
    n = x.shape[0]
    adj3 = adj[0].reshape(_N_STREAMS, n // _N_STREAMS, n)
    h1 = _gcn_layer(x, adj3, W1, b1, a1)
    h2 = _gcn_layer(h1, adj3, W2, b2, a2)
    return h2[None]


# speedup vs baseline: 3.5798x; 1.0075x over previous
import math

import jax
import jax.numpy as jnp
from jax import lax
from jax.experimental import pallas as pl
from jax.experimental.pallas import tpu as pltpu

_N_STREAMS = 2
_BM_FEAT = 1000
_BM_AGG = 200


def _layer_body(x_ref, w_ref, b_ref, al_ref, *rest):
    a_refs = rest[:-2]
    o_ref = rest[-2]
    y_scr = rest[-1]
    i = pl.program_id(0)
    nf = y_scr.shape[0] // x_ref.shape[0]

    @pl.when(i < nf)
    def _feat():
        row = pl.multiple_of(i * x_ref.shape[0], x_ref.shape[0])
        y_scr[pl.ds(row, x_ref.shape[0]), :] = lax.dot_general(
            x_ref[...], w_ref[...], (((1,), (1,)), ((), ())),
            preferred_element_type=jnp.float32)

    @pl.when(i >= nf)
    def _agg():
        alpha = al_ref[0, 0]
        for q, a_ref in enumerate(a_refs):
            acc = lax.dot_general(
                a_ref[0], y_scr[...], (((1,), (0,)), ((), ())),
                preferred_element_type=jnp.float32)
            h = acc + b_ref[...]
            o_ref[q] = jnp.where(h >= 0.0, h, alpha * h).astype(o_ref.dtype)


def _gcn_layer(x, adj3, w, b, alpha, out_dtype=jnp.float32):
    n, d_in = x.shape
    ns, rows_per_stream, _ = adj3.shape
    d_out = w.shape[0]
    nf = n // _BM_FEAT
    na = math.ceil(rows_per_stream / _BM_AGG)
    grid = (nf + na,)

    def _x_map(i):
        return (jnp.minimum(i, nf - 1), 0)

    def _adj_map_for(q):
        def _m(i):
            return (q, jnp.maximum(i - nf, 0), 0)
        return _m

    def _out_map(i):
        return (0, jnp.maximum(i - nf, 0), 0)

    h = pl.pallas_call(
        _layer_body,
        grid=grid,
        in_specs=[
            pl.BlockSpec((_BM_FEAT, d_in), _x_map),
            pl.BlockSpec((d_out, d_in), lambda i: (0, 0)),
            pl.BlockSpec((1, d_out), lambda i: (0, 0)),
            pl.BlockSpec((1, 1), lambda i: (0, 0)),
        ] + [
            pl.BlockSpec((1, _BM_AGG, n), _adj_map_for(q))
            for q in range(ns)
        ],
        out_specs=pl.BlockSpec((ns, _BM_AGG, d_out), _out_map),
        out_shape=jax.ShapeDtypeStruct((ns, rows_per_stream, d_out), out_dtype),
        scratch_shapes=[pltpu.VMEM((n, d_out), jnp.float32)],
    )(x, w, b.reshape(1, -1), alpha.reshape(1, 1), *([adj3] * ns))
    return h.reshape(ns * rows_per_stream, d_out)


def kernel(features, seq1, adj, b1, W1, a1, b2, W2, a2, sparse):
    del seq1, sparse
    x = features[0]
    n = x.shape[0]
    adj3 = adj[0].reshape(_N_STREAMS, n // _N_STREAMS, n)
    h1 = _gcn_layer(x, adj3, W1, b1, a1, out_dtype=jnp.bfloat16)
    h2 = _gcn_layer(h1, adj3, W2.astype(jnp.bfloat16), b2, a2)
    return h2[None]
